# Initial kernel scaffold; baseline (speedup 1.0000x reference)
import jax
import jax.numpy as jnp
from jax.experimental import pallas as pl


def kernel(edge_index, z, emb, W0, b0, W1, b1, W2, b2, W3, b3, c1w, c1b, c2w, c2b, l1w, l1b, l2w, l2b):
    raise NotImplementedError("write your pallas kernel here")


# speedup vs baseline: 2.3337x; 2.3337x over previous
import functools

import jax
import jax.numpy as jnp
from jax import lax
from jax.experimental import pallas as pl
from jax.experimental.pallas import tpu as pltpu
from jax.experimental.pallas import tpu_sc as plsc

N = 10000
E = 320000
HID = 128
K = 100
NPAD = 10240
NC = 2
NS = 16
NW = NC * NS
EPW = E // NW
CH = 80
NCHUNK = EPW // CH
RPT = NPAD // NS
GPW = NPAD // NW
ACCR = NPAD + 16
ZR = ACCR // NS
F32 = jnp.float32
I32 = jnp.int32


def _mesh():
    return plsc.VectorSubcoreMesh(core_axis_name="c", subcore_axis_name="s")


def _sc_degrees(src, dst, zflat):
    @functools.partial(
        pl.kernel,
        out_type=(jax.ShapeDtypeStruct((NW, NPAD), F32),
                  jax.ShapeDtypeStruct((NW, NPAD), F32)),
        mesh=_mesh(),
        compiler_params=pltpu.CompilerParams(needs_layout_passes=False),
        scratch_types=[
            pltpu.VMEM((NPAD,), F32),
            pltpu.VMEM((NPAD,), F32),
            pltpu.VMEM((CH,), I32),
            pltpu.VMEM((CH,), I32),
        ],
    )
    def k(src_h, dst_h, z_h, outs_h, outd_h, hs, hd, is_v, id_v):
        c = lax.axis_index("c")
        s = lax.axis_index("s")
        wid = s * NC + c
        pltpu.sync_copy(z_h, hs)
        pltpu.sync_copy(z_h, hd)
        base = wid * EPW
        ones = jnp.full((16,), 1.0, F32)

        def body(j, carry):
            off = base + j * CH
            pltpu.sync_copy(src_h.at[pl.ds(off, CH)], is_v)
            pltpu.sync_copy(dst_h.at[pl.ds(off, CH)], id_v)
            for t in range(CH // 16):
                plsc.addupdate_scatter(hs, [is_v[pl.ds(t * 16, 16)]], ones)
                plsc.addupdate_scatter(hd, [id_v[pl.ds(t * 16, 16)]], ones)
            return carry

        lax.fori_loop(0, NCHUNK, body, 0)
        pltpu.sync_copy(hs, outs_h.at[wid])
        pltpu.sync_copy(hd, outd_h.at[wid])

    return k(src, dst, zflat)


def _sc_gather_rows(table, idx):
    @functools.partial(
        pl.kernel,
        out_type=jax.ShapeDtypeStruct((NPAD, HID), F32),
        mesh=_mesh(),
        compiler_params=pltpu.CompilerParams(needs_layout_passes=False),
        scratch_types=[
            pltpu.VMEM((CH,), I32),
            pltpu.VMEM((CH, HID), F32),
            pltpu.SemaphoreType.DMA,
        ],
    )
    def k(tab_h, idx_h, out_h, idx_v, rows_v, sem):
        c = lax.axis_index("c")
        s = lax.axis_index("s")
        wid = s * NC + c
        base = wid * GPW

        def body(j, carry):
            off = base + j * CH
            pltpu.sync_copy(idx_h.at[pl.ds(off, CH)], idx_v)
            pltpu.async_copy(tab_h.at[idx_v], rows_v, sem).wait()
            pltpu.sync_copy(rows_v, out_h.at[pl.ds(off, CH)])
            return carry

        lax.fori_loop(0, GPW // CH, body, 0)

    return k(table, idx)


def _sc_mp128(table, ssrc, sdst, zrows):
    @functools.partial(
        pl.kernel,
        out_type=jax.ShapeDtypeStruct((NC, NPAD, HID), F32),
        mesh=_mesh(),
        compiler_params=pltpu.CompilerParams(needs_layout_passes=False),
        scratch_types=[
            pltpu.VMEM_SHARED((ACCR, HID), F32),
            pltpu.VMEM((96,), I32),
            pltpu.VMEM((CH,), I32),
            pltpu.VMEM((CH, HID), F32),
            pltpu.VMEM((16, HID), F32),
            pltpu.SemaphoreType.DMA,
        ],
    )
    def k(tab_h, src_h, dst_h, z_h, out_h, acc, idv, isv, rows_v, stag, sem):
        c = lax.axis_index("c")
        s = lax.axis_index("s")
        pltpu.sync_copy(z_h, acc.at[pl.ds(s * ZR, ZR)])
        plsc.subcore_barrier()

        wlen = jnp.where(s < 11, 10080, jnp.where(s < 15, 9840, 9760))
        woff = (E // 2) * c + jnp.where(
            s < 11, s * 10080,
            jnp.where(s < 15, 110880 + (s - 11) * 9840, 150240))
        nchunks = wlen // CH
        lane = lax.iota(I32, 16)

        def chunk_body(q, st):
            cur, p, ids, a = st[0], st[1], st[2], st[3:]
            off = woff + q * CH
            pltpu.sync_copy(dst_h.at[pl.ds(off, CH)], idv.at[pl.ds(0, CH)])
            pltpu.sync_copy(src_h.at[pl.ds(off, CH)], isv)
            pltpu.async_copy(tab_h.at[isv], rows_v, sem).wait()

            def edge(j, st2):
                cur, p, ids, a = st2[0], st2[1], st2[2], st2[3:]
                nid = idv[pl.ds(j, 16)][0]
                row = tuple(rows_v[j, pl.ds(16 * cc, 16)] for cc in range(8))
                same = nid == cur

                def on_new(op):
                    cur_, p_, ids_ = op
                    cid = jnp.where(cur_ < 0, NPAD, cur_)
                    ids2 = jnp.where(lane == p_, cid, ids_)
                    for cc in range(8):
                        stag[p_, pl.ds(16 * cc, 16)] = a[cc]

                    @pl.when(p_ == 15)
                    def _():
                        pltpu.sync_copy(stag, acc.at[ids2], add=True)

                    return jnp.where(p_ == 15, 0, p_ + 1), ids2

                def on_same(op):
                    return op[1], op[2]

                p2, ids2 = lax.cond(same, on_same, on_new, (cur, p, ids))
                a2 = tuple(jnp.where(same, a[cc] + row[cc], row[cc])
                           for cc in range(8))
                return (nid, p2, ids2) + a2

            st2 = lax.fori_loop(0, CH, edge, (cur, p, ids) + a)
            return st2

        zero8 = tuple(jnp.zeros((16,), F32) for _ in range(8))
        st = (jnp.int32(-1), jnp.int32(0), jnp.full((16,), NPAD, I32)) + zero8
        st = lax.fori_loop(0, nchunks, chunk_body, st)
        cur, p, ids, a = st[0], st[1], st[2], st[3:]
        cid = jnp.where(cur < 0, NPAD, cur)
        ids2 = jnp.where(lane == p, cid, ids)
        for cc in range(8):
            stag[p, pl.ds(16 * cc, 16)] = a[cc]
        p2 = p + 1
        ids3 = jnp.where(lane < p2, ids2, NPAD + lane)
        pltpu.sync_copy(stag, acc.at[ids3], add=True)

        plsc.subcore_barrier()
        pltpu.sync_copy(acc.at[pl.ds(s * 640, 640)],
                        out_h.at[c, pl.ds(s * 640, 640)])

    return k(table, ssrc, sdst, zrows)


def _tc_norms(hs, hd):
    def body(hs_r, hd_r, ns_r, nd_r):
        ns_r[...] = lax.rsqrt(jnp.maximum(jnp.sum(hs_r[...], axis=0), 1.0))
        nd_r[...] = lax.rsqrt(jnp.maximum(jnp.sum(hd_r[...], axis=0), 1.0))

    return pl.pallas_call(
        body,
        out_shape=(jax.ShapeDtypeStruct((NPAD // HID, HID), F32),
                   jax.ShapeDtypeStruct((NPAD // HID, HID), F32)),
    )(hs, hd)


def _tc_scale(g, ns_col):
    def body(g_r, ns_r, o_r):
        o_r[...] = g_r[...] * ns_r[...]

    return pl.pallas_call(
        body, out_shape=jax.ShapeDtypeStruct((NPAD, HID), F32),
    )(g, ns_col)


def _tc_fuse(acc, nd_col, W, b_row, ns_col):
    def body(a_r, nd_r, w_r, b_r, ns_r, h_r, t_r):
        agg = (a_r[0] + a_r[1]) * nd_r[...]
        h = jnp.tanh(jnp.dot(agg, w_r[...], preferred_element_type=F32)
                     + b_r[...])
        h_r[...] = h
        t_r[...] = h * ns_r[...]

    return pl.pallas_call(
        body,
        out_shape=(jax.ShapeDtypeStruct((NPAD, HID), F32),
                   jax.ShapeDtypeStruct((NPAD, HID), F32)),
    )(acc, nd_col, W, b_row, ns_col)


def _tc_h3(acc, nd_col, W3, b3_11):
    def body(a_r, nd_r, w_r, b_r, h_r):
        agg = (a_r[0] + a_r[1]) * nd_r[...]
        h_r[...] = jnp.tanh(jnp.dot(agg, w_r[...], preferred_element_type=F32)
                            + b_r[0, 0])

    return pl.pallas_call(
        body, out_shape=jax.ShapeDtypeStruct((NPAD, 1), F32),
    )(acc, nd_col, W3, b3_11)


def _tc_topk(h380):
    def body(h3_r, idx_r):
        li = (lax.broadcasted_iota(I32, (NPAD // HID, HID), 0) * HID
              + lax.broadcasted_iota(I32, (NPAD // HID, HID), 1))
        vals = jnp.where(li < N, h3_r[...], -jnp.inf)

        def step(k, carry):
            v, idxv = carry
            m = jnp.max(v)
            i = jnp.min(jnp.where(v == m, li, jnp.full_like(li, NPAD)))
            idxv = jnp.where(
                lax.broadcasted_iota(I32, (1, HID), 1) == k, i, idxv)
            v = jnp.where(li == i, -jnp.inf, v)
            return v, idxv

        _, idxv = lax.fori_loop(0, K, step, (vals, jnp.zeros((1, HID), I32)))
        idx_r[...] = idxv

    return pl.pallas_call(
        body, out_shape=jax.ShapeDtypeStruct((1, HID), I32),
    )(h380)


def _tc_readout(idx, h0, h1, h2, h3c, c1m, c1b2, w2m, c2b2, l1w, l1b2,
                l2wT, l2b2):
    def body(idx_r, h0_r, h1_r, h2_r, h3_r, c1m_r, c1b_r, w2m_r, c2b_r,
             l1_r, l1b_r, l2w_r, l2b_r, out_r, P, V):
        for k in range(K):
            i = idx_r[k]
            slot = (k // 2) if k % 2 == 0 else (K // 2 + k // 2)
            P[pl.ds(slot, 1), pl.ds(0, HID)] = h0_r[pl.ds(i, 1), :]
            P[pl.ds(slot, 1), pl.ds(HID, HID)] = h1_r[pl.ds(i, 1), :]
            P[pl.ds(slot, 1), pl.ds(2 * HID, HID)] = h2_r[pl.ds(i, 1), :]
            P[pl.ds(slot, 1), pl.ds(3 * HID, 1)] = h3_r[pl.ds(i, 1), :]
        z1 = jnp.maximum(jnp.dot(P[...], c1m_r[...],
                                 preferred_element_type=F32) + c1b_r[...], 0.0)
        hp = jnp.maximum(z1[0:K // 2], z1[K // 2:K])
        xc = jnp.concatenate([hp[k:46 + k] for k in range(5)], axis=1)
        o2 = jnp.maximum(jnp.dot(xc, w2m_r[...],
                                 preferred_element_type=F32) + c2b_r[...], 0.0)
        o2t = jnp.transpose(o2)
        for o in range(32):
            V[pl.ds(0, 1), pl.ds(o * 46, 46)] = o2t[o:o + 1, :]
        hl = jnp.maximum(jnp.dot(V[...], l1_r[...],
                                 preferred_element_type=F32) + l1b_r[...], 0.0)
        out_r[...] = (jnp.sum(hl * l2w_r[...], axis=1, keepdims=True)
                      + l2b_r[...])

    in_specs = [pl.BlockSpec(memory_space=pltpu.MemorySpace.SMEM)]
    in_specs += [pl.BlockSpec(memory_space=pltpu.MemorySpace.VMEM)] * 12
    return pl.pallas_call(
        body,
        in_specs=in_specs,
        out_shape=jax.ShapeDtypeStruct((1, 1), F32),
        scratch_shapes=[pltpu.VMEM((K, 385), F32),
                        pltpu.VMEM((1, 1472), F32)],
    )(idx, h0, h1, h2, h3c, c1m, c1b2, w2m, c2b2, l1w, l1b2, l2wT, l2b2)


def kernel(edge_index, z, emb, W0, b0, W1, b1, W2, b2, W3, b3,
           c1w, c1b, c2w, c2b, l1w, l1b, l2w, l2b):
    src = edge_index[0]
    dst = edge_index[1]
    order = jnp.argsort(dst, stable=True)
    ssrc = src[order]
    sdst = dst[order]
    zflat = jnp.zeros((NPAD,), F32)
    zrows = jnp.zeros((ZR, HID), F32)
    z_pad = jnp.concatenate([z, jnp.zeros((NPAD - N,), I32)])

    hs_p, hd_p = _sc_degrees(src, dst, zflat)
    ns80, nd80 = _tc_norms(
        hs_p.reshape(NW, NPAD // HID, HID), hd_p.reshape(NW, NPAD // HID, HID))
    ns_col = ns80.reshape(NPAD, 1)
    nd_col = nd80.reshape(NPAD, 1)

    g = _sc_gather_rows(emb, z_pad)
    t = _tc_scale(g, ns_col)
    acc = _sc_mp128(t, ssrc, sdst, zrows)
    h0, t = _tc_fuse(acc, nd_col, W0, b0.reshape(1, HID), ns_col)
    acc = _sc_mp128(t, ssrc, sdst, zrows)
    h1, t = _tc_fuse(acc, nd_col, W1, b1.reshape(1, HID), ns_col)
    acc = _sc_mp128(t, ssrc, sdst, zrows)
    h2, t = _tc_fuse(acc, nd_col, W2, b2.reshape(1, HID), ns_col)
    acc = _sc_mp128(t, ssrc, sdst, zrows)
    h3c = _tc_h3(acc, nd_col, W3, b3.reshape(1, 1))

    idx = _tc_topk(h3c.reshape(NPAD // HID, HID))

    c1m = c1w[:, 0, :].T
    w2m = jnp.transpose(c2w, (2, 1, 0)).reshape(80, 32)
    out = _tc_readout(
        idx.reshape(HID), h0, h1, h2, h3c,
        c1m, c1b.reshape(1, 16), w2m, c2b.reshape(1, 32),
        l1w, l1b.reshape(1, HID), l2w.reshape(1, HID), l2b.reshape(1, 1))
    return out


# speedup vs baseline: 3.3607x; 1.4401x over previous
import functools

import jax
import jax.numpy as jnp
from jax import lax
from jax.experimental import pallas as pl
from jax.experimental.pallas import tpu as pltpu
from jax.experimental.pallas import tpu_sc as plsc

N = 10000
E = 320000
HID = 128
K = 100
NPAD = 10240
NC = 2
NS = 16
NW = NC * NS
EPW = E // NW
CH = 80
NCHUNK = EPW // CH
RPT = NPAD // NS
GPW = NPAD // NW
ACCR = NPAD + 16
ZR = ACCR // NS
F32 = jnp.float32
I32 = jnp.int32


def _mesh():
    return plsc.VectorSubcoreMesh(core_axis_name="c", subcore_axis_name="s")


def _sc_degrees(src, dst, zflat):
    @functools.partial(
        pl.kernel,
        out_type=(jax.ShapeDtypeStruct((NW, NPAD), F32),
                  jax.ShapeDtypeStruct((NW, NPAD), F32)),
        mesh=_mesh(),
        compiler_params=pltpu.CompilerParams(needs_layout_passes=False),
        scratch_types=[
            pltpu.VMEM((NPAD,), F32),
            pltpu.VMEM((NPAD,), F32),
            pltpu.VMEM((2000,), I32),
            pltpu.VMEM((2000,), I32),
        ],
    )
    def k(src_h, dst_h, z_h, outs_h, outd_h, hs, hd, is_v, id_v):
        c = lax.axis_index("c")
        s = lax.axis_index("s")
        wid = s * NC + c
        pltpu.sync_copy(z_h, hs)
        pltpu.sync_copy(z_h, hd)
        base = wid * EPW
        ones = jnp.full((16,), 1.0, F32)

        def body(j, carry):
            off = base + j * 2000
            pltpu.sync_copy(src_h.at[pl.ds(off, 2000)], is_v)
            pltpu.sync_copy(dst_h.at[pl.ds(off, 2000)], id_v)
            def grp(t, cc):
                plsc.addupdate_scatter(hs, [is_v[pl.ds(t * 16, 16)]], ones)
                plsc.addupdate_scatter(hd, [id_v[pl.ds(t * 16, 16)]], ones)
                return cc
            lax.fori_loop(0, 125, grp, 0)
            return carry

        lax.fori_loop(0, EPW // 2000, body, 0)
        pltpu.sync_copy(hs, outs_h.at[wid])
        pltpu.sync_copy(hd, outd_h.at[wid])

    return k(src, dst, zflat)


def _sc_gather_rows(table, idx):
    @functools.partial(
        pl.kernel,
        out_type=jax.ShapeDtypeStruct((NPAD, HID), F32),
        mesh=_mesh(),
        compiler_params=pltpu.CompilerParams(needs_layout_passes=False),
        scratch_types=[
            pltpu.VMEM((CH,), I32),
            pltpu.VMEM((CH, HID), F32),
            pltpu.SemaphoreType.DMA,
        ],
    )
    def k(tab_h, idx_h, out_h, idx_v, rows_v, sem):
        c = lax.axis_index("c")
        s = lax.axis_index("s")
        wid = s * NC + c
        base = wid * GPW

        def body(j, carry):
            off = base + j * CH
            pltpu.sync_copy(idx_h.at[pl.ds(off, CH)], idx_v)
            pltpu.async_copy(tab_h.at[idx_v], rows_v, sem).wait()
            pltpu.sync_copy(rows_v, out_h.at[pl.ds(off, CH)])
            return carry

        lax.fori_loop(0, GPW // CH, body, 0)

    return k(table, idx)


def _sc_mp128(table, ssrc, sdst, zrows):
    @functools.partial(
        pl.kernel,
        out_type=jax.ShapeDtypeStruct((NC, NPAD, HID), F32),
        mesh=_mesh(),
        compiler_params=pltpu.CompilerParams(needs_layout_passes=False),
        scratch_types=[
            pltpu.VMEM_SHARED((ACCR, HID), F32),
            pltpu.VMEM((320,), I32),
            pltpu.VMEM((320,), I32),
            pltpu.VMEM((320, HID), F32),
            pltpu.VMEM((16, HID), F32),
            pltpu.SemaphoreType.DMA,
        ],
    )
    def k(tab_h, src_h, dst_h, z_h, out_h, acc, idv, isv, rows_v, stag, sem):
        c = lax.axis_index("c")
        s = lax.axis_index("s")
        pltpu.sync_copy(z_h, acc.at[pl.ds(s * ZR, ZR)])
        plsc.subcore_barrier()

        wlen = jnp.where(s < 11, 10080, jnp.where(s < 15, 9840, 9760))
        woff = (E // 2) * c + jnp.where(
            s < 11, s * 10080,
            jnp.where(s < 15, 110880 + (s - 11) * 9840, 150240))
        nsuper = wlen // 320
        ntail = (wlen - nsuper * 320) // CH
        lane = lax.iota(I32, 16)

        def scan_groups(ngroups, st):
            def group(g, st2):
                d16 = idv[pl.ds(16 * g, 16)]
                st3 = st2
                for t in range(16):
                    cur, p, ids, a = st3[0], st3[1], st3[2], st3[3:]
                    nid = d16[t]
                    row = tuple(rows_v[16 * g + t, pl.ds(16 * cc, 16)]
                                for cc in range(8))
                    same = nid == cur
                    notsame = jnp.logical_not(same)
                    for cc in range(8):
                        stag[p, pl.ds(16 * cc, 16)] = a[cc]
                    cid = jnp.where(cur < 0, NPAD, cur)
                    ids2 = jnp.where(
                        jnp.logical_and(lane == p, notsame), cid, ids)

                    @pl.when(jnp.logical_and(notsame, p == 15))
                    def _():
                        pltpu.sync_copy(stag, acc.at[ids2], add=True)

                    p2 = jnp.where(notsame,
                                   jnp.where(p == 15, 0, p + 1), p)
                    a2 = tuple(jnp.where(same, a[cc] + row[cc], row[cc])
                               for cc in range(8))
                    st3 = (nid, p2, ids2) + a2
                return st3
            return lax.fori_loop(0, ngroups, group, st)

        def super_body(q, st):
            off = woff + q * 320
            pltpu.sync_copy(dst_h.at[pl.ds(off, 320)], idv)
            pltpu.sync_copy(src_h.at[pl.ds(off, 320)], isv)
            descs = [pltpu.async_copy(
                tab_h.at[isv.at[pl.ds(i * CH, CH)]],
                rows_v.at[pl.ds(i * CH, CH)], sem) for i in range(4)]
            for d in descs:
                d.wait()
            return scan_groups(20, st)

        def tail_body(q, st):
            off = woff + nsuper * 320 + q * CH
            pltpu.sync_copy(dst_h.at[pl.ds(off, CH)], idv.at[pl.ds(0, CH)])
            pltpu.sync_copy(src_h.at[pl.ds(off, CH)], isv.at[pl.ds(0, CH)])
            pltpu.async_copy(tab_h.at[isv.at[pl.ds(0, CH)]],
                             rows_v.at[pl.ds(0, CH)], sem).wait()
            return scan_groups(5, st)

        zero8 = tuple(jnp.zeros((16,), F32) for _ in range(8))
        st = (jnp.int32(-1), jnp.int32(0), jnp.full((16,), NPAD, I32)) + zero8
        st = lax.fori_loop(0, nsuper, super_body, st)
        st = lax.fori_loop(0, ntail, tail_body, st)
        cur, p, ids, a = st[0], st[1], st[2], st[3:]
        cid = jnp.where(cur < 0, NPAD, cur)
        ids2 = jnp.where(lane == p, cid, ids)
        for cc in range(8):
            stag[p, pl.ds(16 * cc, 16)] = a[cc]
        ids3 = jnp.where(lane < p + 1, ids2, NPAD + lane)
        pltpu.sync_copy(stag, acc.at[ids3], add=True)

        plsc.subcore_barrier()
        pltpu.sync_copy(acc.at[pl.ds(s * 640, 640)],
                        out_h.at[c, pl.ds(s * 640, 640)])

    return k(table, ssrc, sdst, zrows)


def _tc_norms(hs, hd):
    def body(hs_r, hd_r, ns_r, nd_r):
        ns_r[...] = lax.rsqrt(jnp.maximum(jnp.sum(hs_r[...], axis=0), 1.0))
        nd_r[...] = lax.rsqrt(jnp.maximum(jnp.sum(hd_r[...], axis=0), 1.0))

    return pl.pallas_call(
        body,
        out_shape=(jax.ShapeDtypeStruct((NPAD // HID, HID), F32),
                   jax.ShapeDtypeStruct((NPAD // HID, HID), F32)),
    )(hs, hd)


def _tc_scale(g, ns_col):
    def body(g_r, ns_r, o_r):
        o_r[...] = g_r[...] * ns_r[...]

    return pl.pallas_call(
        body, out_shape=jax.ShapeDtypeStruct((NPAD, HID), F32),
    )(g, ns_col)


def _tc_fuse(acc, nd_col, W, b_row, ns_col):
    def body(a_r, nd_r, w_r, b_r, ns_r, h_r, t_r):
        agg = (a_r[0] + a_r[1]) * nd_r[...]
        h = jnp.tanh(jnp.dot(agg, w_r[...], preferred_element_type=F32)
                     + b_r[...])
        h_r[...] = h
        t_r[...] = h * ns_r[...]

    return pl.pallas_call(
        body,
        out_shape=(jax.ShapeDtypeStruct((NPAD, HID), F32),
                   jax.ShapeDtypeStruct((NPAD, HID), F32)),
    )(acc, nd_col, W, b_row, ns_col)


def _tc_h3(acc, nd_col, W3, b3_11):
    def body(a_r, nd_r, w_r, b_r, h_r):
        agg = (a_r[0] + a_r[1]) * nd_r[...]
        h_r[...] = jnp.tanh(jnp.dot(agg, w_r[...], preferred_element_type=F32)
                            + b_r[0, 0])

    return pl.pallas_call(
        body, out_shape=jax.ShapeDtypeStruct((NPAD, 1), F32),
    )(acc, nd_col, W3, b3_11)


def _tc_topk(h380):
    def body(h3_r, idx_r):
        li = (lax.broadcasted_iota(I32, (NPAD // HID, HID), 0) * HID
              + lax.broadcasted_iota(I32, (NPAD // HID, HID), 1))
        vals = jnp.where(li < N, h3_r[...], -jnp.inf)

        def step(k, carry):
            v, idxv = carry
            m = jnp.max(v)
            i = jnp.min(jnp.where(v == m, li, jnp.full_like(li, NPAD)))
            idxv = jnp.where(
                lax.broadcasted_iota(I32, (1, HID), 1) == k, i, idxv)
            v = jnp.where(li == i, -jnp.inf, v)
            return v, idxv

        _, idxv = lax.fori_loop(0, K, step, (vals, jnp.zeros((1, HID), I32)))
        idx_r[...] = idxv

    return pl.pallas_call(
        body, out_shape=jax.ShapeDtypeStruct((1, HID), I32),
    )(h380)


def _tc_readout(idx, h0, h1, h2, h3c, c1m, c1b2, w2m, c2b2, l1w, l1b2,
                l2wT, l2b2):
    def body(idx_r, h0_r, h1_r, h2_r, h3_r, c1m_r, c1b_r, w2m_r, c2b_r,
             l1_r, l1b_r, l2w_r, l2b_r, out_r, P, V):
        for k in range(K):
            i = idx_r[k]
            slot = (k // 2) if k % 2 == 0 else (K // 2 + k // 2)
            P[pl.ds(slot, 1), pl.ds(0, HID)] = h0_r[pl.ds(i, 1), :]
            P[pl.ds(slot, 1), pl.ds(HID, HID)] = h1_r[pl.ds(i, 1), :]
            P[pl.ds(slot, 1), pl.ds(2 * HID, HID)] = h2_r[pl.ds(i, 1), :]
            P[pl.ds(slot, 1), pl.ds(3 * HID, 1)] = h3_r[pl.ds(i, 1), :]
        z1 = jnp.maximum(jnp.dot(P[...], c1m_r[...],
                                 preferred_element_type=F32) + c1b_r[...], 0.0)
        hp = jnp.maximum(z1[0:K // 2], z1[K // 2:K])
        xc = jnp.concatenate([hp[k:46 + k] for k in range(5)], axis=1)
        o2 = jnp.maximum(jnp.dot(xc, w2m_r[...],
                                 preferred_element_type=F32) + c2b_r[...], 0.0)
        o2t = jnp.transpose(o2)
        for o in range(32):
            V[pl.ds(0, 1), pl.ds(o * 46, 46)] = o2t[o:o + 1, :]
        hl = jnp.maximum(jnp.dot(V[...], l1_r[...],
                                 preferred_element_type=F32) + l1b_r[...], 0.0)
        out_r[...] = (jnp.sum(hl * l2w_r[...], axis=1, keepdims=True)
                      + l2b_r[...])

    in_specs = [pl.BlockSpec(memory_space=pltpu.MemorySpace.SMEM)]
    in_specs += [pl.BlockSpec(memory_space=pltpu.MemorySpace.VMEM)] * 12
    return pl.pallas_call(
        body,
        in_specs=in_specs,
        out_shape=jax.ShapeDtypeStruct((1, 1), F32),
        scratch_shapes=[pltpu.VMEM((K, 385), F32),
                        pltpu.VMEM((1, 1472), F32)],
    )(idx, h0, h1, h2, h3c, c1m, c1b2, w2m, c2b2, l1w, l1b2, l2wT, l2b2)


def kernel(edge_index, z, emb, W0, b0, W1, b1, W2, b2, W3, b3,
           c1w, c1b, c2w, c2b, l1w, l1b, l2w, l2b):
    src = edge_index[0]
    dst = edge_index[1]
    order = jnp.argsort(dst, stable=True)
    ssrc = src[order]
    sdst = dst[order]
    zflat = jnp.zeros((NPAD,), F32)
    zrows = jnp.zeros((ZR, HID), F32)
    z_pad = jnp.concatenate([z, jnp.zeros((NPAD - N,), I32)])

    hs_p, hd_p = _sc_degrees(src, dst, zflat)
    ns80, nd80 = _tc_norms(
        hs_p.reshape(NW, NPAD // HID, HID), hd_p.reshape(NW, NPAD // HID, HID))
    ns_col = ns80.reshape(NPAD, 1)
    nd_col = nd80.reshape(NPAD, 1)

    g = _sc_gather_rows(emb, z_pad)
    t = _tc_scale(g, ns_col)
    acc = _sc_mp128(t, ssrc, sdst, zrows)
    h0, t = _tc_fuse(acc, nd_col, W0, b0.reshape(1, HID), ns_col)
    acc = _sc_mp128(t, ssrc, sdst, zrows)
    h1, t = _tc_fuse(acc, nd_col, W1, b1.reshape(1, HID), ns_col)
    acc = _sc_mp128(t, ssrc, sdst, zrows)
    h2, t = _tc_fuse(acc, nd_col, W2, b2.reshape(1, HID), ns_col)
    acc = _sc_mp128(t, ssrc, sdst, zrows)
    h3c = _tc_h3(acc, nd_col, W3, b3.reshape(1, 1))

    idx = _tc_topk(h3c.reshape(NPAD // HID, HID))

    c1m = c1w[:, 0, :].T
    w2m = jnp.transpose(c2w, (2, 1, 0)).reshape(80, 32)
    out = _tc_readout(
        idx.reshape(HID), h0, h1, h2, h3c,
        c1m, c1b.reshape(1, 16), w2m, c2b.reshape(1, 32),
        l1w, l1b.reshape(1, HID), l2w.reshape(1, HID), l2b.reshape(1, 1))
    return out


# speedup vs baseline: 3.6586x; 1.0886x over previous
import functools

import jax
import jax.numpy as jnp
from jax import lax
from jax.experimental import pallas as pl
from jax.experimental.pallas import tpu as pltpu
from jax.experimental.pallas import tpu_sc as plsc

N = 10000
E = 320000
HID = 128
K = 100
NPAD = 10240
NC = 2
NS = 16
NW = NC * NS
EPW = E // NW
CH = 80
NCHUNK = EPW // CH
RPT = NPAD // NS
GPW = NPAD // NW
ACCR = NPAD + 16
ZR = ACCR // NS
F32 = jnp.float32
I32 = jnp.int32


def _mesh():
    return plsc.VectorSubcoreMesh(core_axis_name="c", subcore_axis_name="s")


def _sc_degrees(src, dst, zflat):
    @functools.partial(
        pl.kernel,
        out_type=(jax.ShapeDtypeStruct((NW, NPAD), F32),
                  jax.ShapeDtypeStruct((NW, NPAD), F32)),
        mesh=_mesh(),
        compiler_params=pltpu.CompilerParams(needs_layout_passes=False),
        scratch_types=[
            pltpu.VMEM((NPAD,), F32),
            pltpu.VMEM((NPAD,), F32),
            pltpu.VMEM((2000,), I32),
            pltpu.VMEM((2000,), I32),
        ],
    )
    def k(src_h, dst_h, z_h, outs_h, outd_h, hs, hd, is_v, id_v):
        c = lax.axis_index("c")
        s = lax.axis_index("s")
        wid = s * NC + c
        pltpu.sync_copy(z_h, hs)
        pltpu.sync_copy(z_h, hd)
        base = wid * EPW
        ones = jnp.full((16,), 1.0, F32)

        def body(j, carry):
            off = base + j * 2000
            pltpu.sync_copy(src_h.at[pl.ds(off, 2000)], is_v)
            pltpu.sync_copy(dst_h.at[pl.ds(off, 2000)], id_v)
            def grp(t, cc):
                plsc.addupdate_scatter(hs, [is_v[pl.ds(t * 16, 16)]], ones)
                plsc.addupdate_scatter(hd, [id_v[pl.ds(t * 16, 16)]], ones)
                return cc
            lax.fori_loop(0, 125, grp, 0)
            return carry

        lax.fori_loop(0, EPW // 2000, body, 0)
        pltpu.sync_copy(hs, outs_h.at[wid])
        pltpu.sync_copy(hd, outd_h.at[wid])

    return k(src, dst, zflat)


def _sc_gather_rows(table, idx):
    @functools.partial(
        pl.kernel,
        out_type=jax.ShapeDtypeStruct((NPAD, HID), F32),
        mesh=_mesh(),
        compiler_params=pltpu.CompilerParams(needs_layout_passes=False),
        scratch_types=[
            pltpu.VMEM((CH,), I32),
            pltpu.VMEM((CH, HID), F32),
            pltpu.SemaphoreType.DMA,
        ],
    )
    def k(tab_h, idx_h, out_h, idx_v, rows_v, sem):
        c = lax.axis_index("c")
        s = lax.axis_index("s")
        wid = s * NC + c
        base = wid * GPW

        def body(j, carry):
            off = base + j * CH
            pltpu.sync_copy(idx_h.at[pl.ds(off, CH)], idx_v)
            pltpu.async_copy(tab_h.at[idx_v], rows_v, sem).wait()
            pltpu.sync_copy(rows_v, out_h.at[pl.ds(off, CH)])
            return carry

        lax.fori_loop(0, GPW // CH, body, 0)

    return k(table, idx)


def _sc_mp128(table, ssrc, sdst, zrows):
    @functools.partial(
        pl.kernel,
        out_type=jax.ShapeDtypeStruct((NC, NPAD, HID), F32),
        mesh=_mesh(),
        compiler_params=pltpu.CompilerParams(needs_layout_passes=False),
        scratch_types=[
            pltpu.VMEM_SHARED((ACCR, HID), F32),
            pltpu.VMEM((320,), I32),
            pltpu.VMEM((320,), I32),
            pltpu.VMEM((320, HID), F32),
            pltpu.VMEM((16, HID), F32),
            pltpu.SemaphoreType.DMA,
        ],
    )
    def k(tab_h, src_h, dst_h, z_h, out_h, acc, idv, isv, rows_v, stag, sem):
        c = lax.axis_index("c")
        s = lax.axis_index("s")
        pltpu.sync_copy(z_h, acc.at[pl.ds(s * ZR, ZR)])
        plsc.subcore_barrier()

        wlen = jnp.where(s < 11, 10080, jnp.where(s < 15, 9840, 9760))
        woff = (E // 2) * c + jnp.where(
            s < 11, s * 10080,
            jnp.where(s < 15, 110880 + (s - 11) * 9840, 150240))
        nsuper = wlen // 320
        ntail = (wlen - nsuper * 320) // CH
        lane = lax.iota(I32, 16)

        def scan_groups(glo, ghi, st):
            def group(g, st2):
                d16 = idv[pl.ds(16 * g, 16)]
                st3 = st2
                for t in range(16):
                    cur, p, ids, a = st3[0], st3[1], st3[2], st3[3:]
                    nid = d16[t]
                    row = tuple(rows_v[16 * g + t, pl.ds(16 * cc, 16)]
                                for cc in range(8))
                    same = nid == cur
                    notsame = jnp.logical_not(same)
                    for cc in range(8):
                        stag[p, pl.ds(16 * cc, 16)] = a[cc]
                    cid = jnp.where(cur < 0, NPAD, cur)
                    ids2 = jnp.where(
                        jnp.logical_and(lane == p, notsame), cid, ids)

                    @pl.when(jnp.logical_and(notsame, p == 15))
                    def _():
                        pltpu.sync_copy(stag, acc.at[ids2], add=True)

                    p2 = jnp.where(notsame,
                                   jnp.where(p == 15, 0, p + 1), p)
                    a2 = tuple(jnp.where(same, a[cc] + row[cc], row[cc])
                               for cc in range(8))
                    st3 = (nid, p2, ids2) + a2
                return st3
            return lax.fori_loop(glo, ghi, group, st)

        def super_body(q, st):
            off = woff + q * 320
            pltpu.sync_copy(dst_h.at[pl.ds(off, 320)], idv)
            pltpu.sync_copy(src_h.at[pl.ds(off, 320)], isv)
            descs = [pltpu.async_copy(
                tab_h.at[isv.at[pl.ds(i * CH, CH)]],
                rows_v.at[pl.ds(i * CH, CH)], sem) for i in range(4)]
            for i in range(4):
                descs[i].wait()
                st = scan_groups(5 * i, 5 * (i + 1), st)
            return st

        def tail_body(q, st):
            off = woff + nsuper * 320 + q * CH
            pltpu.sync_copy(dst_h.at[pl.ds(off, CH)], idv.at[pl.ds(0, CH)])
            pltpu.sync_copy(src_h.at[pl.ds(off, CH)], isv.at[pl.ds(0, CH)])
            pltpu.async_copy(tab_h.at[isv.at[pl.ds(0, CH)]],
                             rows_v.at[pl.ds(0, CH)], sem).wait()
            return scan_groups(0, 5, st)

        zero8 = tuple(jnp.zeros((16,), F32) for _ in range(8))
        st = (jnp.int32(-1), jnp.int32(0), jnp.full((16,), NPAD, I32)) + zero8
        st = lax.fori_loop(0, nsuper, super_body, st)
        st = lax.fori_loop(0, ntail, tail_body, st)
        cur, p, ids, a = st[0], st[1], st[2], st[3:]
        cid = jnp.where(cur < 0, NPAD, cur)
        ids2 = jnp.where(lane == p, cid, ids)
        for cc in range(8):
            stag[p, pl.ds(16 * cc, 16)] = a[cc]
        ids3 = jnp.where(lane < p + 1, ids2, NPAD + lane)
        pltpu.sync_copy(stag, acc.at[ids3], add=True)

        plsc.subcore_barrier()
        pltpu.sync_copy(acc.at[pl.ds(s * 640, 640)],
                        out_h.at[c, pl.ds(s * 640, 640)])

    return k(table, ssrc, sdst, zrows)


def _tc_norms(hs, hd):
    def body(hs_r, hd_r, ns_r, nd_r):
        ns_r[...] = lax.rsqrt(jnp.maximum(jnp.sum(hs_r[...], axis=0), 1.0))
        nd_r[...] = lax.rsqrt(jnp.maximum(jnp.sum(hd_r[...], axis=0), 1.0))

    return pl.pallas_call(
        body,
        out_shape=(jax.ShapeDtypeStruct((NPAD // HID, HID), F32),
                   jax.ShapeDtypeStruct((NPAD // HID, HID), F32)),
    )(hs, hd)


def _tc_scale(g, ns_col):
    def body(g_r, ns_r, o_r):
        o_r[...] = g_r[...] * ns_r[...]

    return pl.pallas_call(
        body, out_shape=jax.ShapeDtypeStruct((NPAD, HID), F32),
    )(g, ns_col)


def _tc_fuse(acc, nd_col, W, b_row, ns_col):
    def body(a_r, nd_r, w_r, b_r, ns_r, h_r, t_r):
        agg = (a_r[0] + a_r[1]) * nd_r[...]
        h = jnp.tanh(jnp.dot(agg, w_r[...], preferred_element_type=F32)
                     + b_r[...])
        h_r[...] = h
        t_r[...] = h * ns_r[...]

    return pl.pallas_call(
        body,
        out_shape=(jax.ShapeDtypeStruct((NPAD, HID), F32),
                   jax.ShapeDtypeStruct((NPAD, HID), F32)),
    )(acc, nd_col, W, b_row, ns_col)


def _tc_h3(acc, nd_col, W3, b3_11):
    def body(a_r, nd_r, w_r, b_r, h_r):
        agg = (a_r[0] + a_r[1]) * nd_r[...]
        h_r[...] = jnp.tanh(jnp.dot(agg, w_r[...], preferred_element_type=F32)
                            + b_r[0, 0])

    return pl.pallas_call(
        body, out_shape=jax.ShapeDtypeStruct((NPAD, 1), F32),
    )(acc, nd_col, W3, b3_11)


def _tc_topk(h380):
    def body(h3_r, idx_r):
        li = (lax.broadcasted_iota(I32, (NPAD // HID, HID), 0) * HID
              + lax.broadcasted_iota(I32, (NPAD // HID, HID), 1))
        vals = jnp.where(li < N, h3_r[...], -jnp.inf)

        def step(k, carry):
            v, idxv = carry
            m = jnp.max(v)
            i = jnp.min(jnp.where(v == m, li, jnp.full_like(li, NPAD)))
            idxv = jnp.where(
                lax.broadcasted_iota(I32, (1, HID), 1) == k, i, idxv)
            v = jnp.where(li == i, -jnp.inf, v)
            return v, idxv

        _, idxv = lax.fori_loop(0, K, step, (vals, jnp.zeros((1, HID), I32)))
        idx_r[...] = idxv

    return pl.pallas_call(
        body, out_shape=jax.ShapeDtypeStruct((1, HID), I32),
    )(h380)


def _tc_readout(idx, h0, h1, h2, h3c, c1m, c1b2, w2m, c2b2, l1w, l1b2,
                l2wT, l2b2):
    def body(idx_r, h0_r, h1_r, h2_r, h3_r, c1m_r, c1b_r, w2m_r, c2b_r,
             l1_r, l1b_r, l2w_r, l2b_r, out_r, P, V):
        for k in range(K):
            i = idx_r[k]
            slot = (k // 2) if k % 2 == 0 else (K // 2 + k // 2)
            P[pl.ds(slot, 1), pl.ds(0, HID)] = h0_r[pl.ds(i, 1), :]
            P[pl.ds(slot, 1), pl.ds(HID, HID)] = h1_r[pl.ds(i, 1), :]
            P[pl.ds(slot, 1), pl.ds(2 * HID, HID)] = h2_r[pl.ds(i, 1), :]
            P[pl.ds(slot, 1), pl.ds(3 * HID, 1)] = h3_r[pl.ds(i, 1), :]
        z1 = jnp.maximum(jnp.dot(P[...], c1m_r[...],
                                 preferred_element_type=F32) + c1b_r[...], 0.0)
        hp = jnp.maximum(z1[0:K // 2], z1[K // 2:K])
        xc = jnp.concatenate([hp[k:46 + k] for k in range(5)], axis=1)
        o2 = jnp.maximum(jnp.dot(xc, w2m_r[...],
                                 preferred_element_type=F32) + c2b_r[...], 0.0)
        o2t = jnp.transpose(o2)
        for o in range(32):
            V[pl.ds(0, 1), pl.ds(o * 46, 46)] = o2t[o:o + 1, :]
        hl = jnp.maximum(jnp.dot(V[...], l1_r[...],
                                 preferred_element_type=F32) + l1b_r[...], 0.0)
        out_r[...] = (jnp.sum(hl * l2w_r[...], axis=1, keepdims=True)
                      + l2b_r[...])

    in_specs = [pl.BlockSpec(memory_space=pltpu.MemorySpace.SMEM)]
    in_specs += [pl.BlockSpec(memory_space=pltpu.MemorySpace.VMEM)] * 12
    return pl.pallas_call(
        body,
        in_specs=in_specs,
        out_shape=jax.ShapeDtypeStruct((1, 1), F32),
        scratch_shapes=[pltpu.VMEM((K, 385), F32),
                        pltpu.VMEM((1, 1472), F32)],
    )(idx, h0, h1, h2, h3c, c1m, c1b2, w2m, c2b2, l1w, l1b2, l2wT, l2b2)


def kernel(edge_index, z, emb, W0, b0, W1, b1, W2, b2, W3, b3,
           c1w, c1b, c2w, c2b, l1w, l1b, l2w, l2b):
    src = edge_index[0]
    dst = edge_index[1]
    order = jnp.argsort(dst, stable=True)
    ssrc = src[order]
    sdst = dst[order]
    zflat = jnp.zeros((NPAD,), F32)
    zrows = jnp.zeros((ZR, HID), F32)
    z_pad = jnp.concatenate([z, jnp.zeros((NPAD - N,), I32)])

    hs_p, hd_p = _sc_degrees(src, dst, zflat)
    ns80, nd80 = _tc_norms(
        hs_p.reshape(NW, NPAD // HID, HID), hd_p.reshape(NW, NPAD // HID, HID))
    ns_col = ns80.reshape(NPAD, 1)
    nd_col = nd80.reshape(NPAD, 1)

    g = _sc_gather_rows(emb, z_pad)
    t = _tc_scale(g, ns_col)
    acc = _sc_mp128(t, ssrc, sdst, zrows)
    h0, t = _tc_fuse(acc, nd_col, W0, b0.reshape(1, HID), ns_col)
    acc = _sc_mp128(t, ssrc, sdst, zrows)
    h1, t = _tc_fuse(acc, nd_col, W1, b1.reshape(1, HID), ns_col)
    acc = _sc_mp128(t, ssrc, sdst, zrows)
    h2, t = _tc_fuse(acc, nd_col, W2, b2.reshape(1, HID), ns_col)
    acc = _sc_mp128(t, ssrc, sdst, zrows)
    h3c = _tc_h3(acc, nd_col, W3, b3.reshape(1, 1))

    idx = _tc_topk(h3c.reshape(NPAD // HID, HID))

    c1m = c1w[:, 0, :].T
    w2m = jnp.transpose(c2w, (2, 1, 0)).reshape(80, 32)
    out = _tc_readout(
        idx.reshape(HID), h0, h1, h2, h3c,
        c1m, c1b.reshape(1, 16), w2m, c2b.reshape(1, 32),
        l1w, l1b.reshape(1, HID), l2w.reshape(1, HID), l2b.reshape(1, 1))
    return out
